# baseline (speedup 1.0000x reference)
import functools

import jax
import jax.numpy as jnp
from jax import lax
from jax.experimental import pallas as pl
from jax.experimental.pallas import tpu as pltpu
from jax.experimental.pallas import tpu_sc as plsc

_B, _S, _D = 4, 8192, 768
_NW = 32
_ROWS = _S // _NW
_R = 32
_NCH = _ROWS // _R
_NT = _NCH * _B
_LANES = 16


def _sc_body(x_hbm, pos_hbm, out_hbm, *refs):
    pos_bufs = refs[0:2]
    x_bufs = refs[2:5]
    psems = refs[5:7]
    isems = refs[7:10]
    osems = refs[10:13]
    wid = lax.axis_index("s") * 2 + lax.axis_index("c")
    row0 = wid * _ROWS

    def pos_slab(c):
        return pos_hbm.at[pl.ds(row0 + c * _R, _R)]

    def x_slab(c, b):
        return x_hbm.at[pl.ds(b * _S + row0 + c * _R, _R)]

    def o_slab(c, b):
        return out_hbm.at[pl.ds(b * _S + row0 + c * _R, _R)]

    def compute(xb, pb):
        def rowbody(r, carry):
            def colbody(cc, carry2):
                for u in range(8):
                    sl = pl.ds(cc * 8 * _LANES + u * _LANES, _LANES)
                    plsc.addupdate(xb.at[r, sl], pb[r, sl])
                return carry2

            return lax.fori_loop(0, _D // (8 * _LANES), colbody, carry)

        lax.fori_loop(0, _R, rowbody, 0)

    pltpu.async_copy(pos_slab(0), pos_bufs[0], psems[0])
    pltpu.async_copy(x_slab(0, 0), x_bufs[0], isems[0])
    pltpu.async_copy(x_slab(0, 1), x_bufs[1], isems[1])

    for t in range(_NT):
        c, b = divmod(t, _B)
        if b == 0:
            pltpu.make_async_copy(pos_slab(c), pos_bufs[c % 2], psems[c % 2]).wait()
            if c + 1 < _NCH:
                pltpu.async_copy(
                    pos_slab(c + 1), pos_bufs[(c + 1) % 2], psems[(c + 1) % 2]
                )
        xb = x_bufs[t % 3]
        pltpu.make_async_copy(x_slab(c, b), xb, isems[t % 3]).wait()
        if t + 2 < _NT:
            c2, b2 = divmod(t + 2, _B)
            if t - 1 >= 0:
                cp, bp = divmod(t - 1, _B)
                pltpu.make_async_copy(
                    x_bufs[(t - 1) % 3], o_slab(cp, bp), osems[(t - 1) % 3]
                ).wait()
            pltpu.async_copy(x_slab(c2, b2), x_bufs[(t + 2) % 3], isems[(t + 2) % 3])
        compute(xb, pos_bufs[c % 2])
        pltpu.async_copy(xb, o_slab(c, b), osems[t % 3])

    for t in (_NT - 3, _NT - 2, _NT - 1):
        c, b = divmod(t, _B)
        pltpu.make_async_copy(x_bufs[t % 3], o_slab(c, b), osems[t % 3]).wait()


def kernel(x, position_embeddings):
    B, S, D = x.shape
    xf = x.reshape(B * S, D)
    mesh = plsc.VectorSubcoreMesh(core_axis_name="c", subcore_axis_name="s")
    f = pl.kernel(
        _sc_body,
        mesh=mesh,
        out_type=jax.ShapeDtypeStruct((B * S, D), jnp.float32),
        scratch_types=(
            [pltpu.VMEM((_R, _D), jnp.float32) for _ in range(5)]
            + [pltpu.SemaphoreType.DMA for _ in range(8)]
        ),
    )
    out = f(xf, position_embeddings)
    return out.reshape(B, S, D)


# speedup vs baseline: 2.5245x; 2.5245x over previous
import functools

import jax
import jax.numpy as jnp
from jax import lax
from jax.experimental import pallas as pl
from jax.experimental.pallas import tpu as pltpu
from jax.experimental.pallas import tpu_sc as plsc

_B, _S, _D = 4, 8192, 768
_NW = 32
_ROWS = _S // _NW
_R = 16
_NCH = _ROWS // _R
_LANES = 16
_CPR = _D // _LANES


def _sc_body(x_hbm, pos_hbm, out_hbm, *refs):
    pos_bufs = refs[0:2]
    x_bufs = refs[2:10]
    psems = refs[10:12]
    isems = refs[12:20]
    osems = refs[20:28]
    wid = lax.axis_index("s") * 2 + lax.axis_index("c")
    row0 = wid * _ROWS

    def pos_slab(c):
        return pos_hbm.at[pl.ds(row0 + c * _R, _R)]

    def x_slab(c, b):
        return x_hbm.at[pl.ds(b * _S + row0 + c * _R, _R)]

    def o_slab(c, b):
        return out_hbm.at[pl.ds(b * _S + row0 + c * _R, _R)]

    pltpu.async_copy(pos_slab(0), pos_bufs[0], psems[0])
    for b in range(_B):
        pltpu.async_copy(x_slab(0, b), x_bufs[b], isems[b])

    def do_chunk(c, q):
        nq = 1 - q
        pltpu.make_async_copy(pos_slab(c), pos_bufs[q], psems[q]).wait()

        @pl.when(c + 1 < _NCH)
        def _():
            pltpu.async_copy(pos_slab(c + 1), pos_bufs[nq], psems[nq])

        for b in range(_B):
            @pl.when((c > 0) & (c + 1 < _NCH))
            def _(b=b):
                pltpu.make_async_copy(
                    x_bufs[nq * _B + b], o_slab(c, b), osems[nq * _B + b]
                ).wait()

            @pl.when(c + 1 < _NCH)
            def _(b=b):
                pltpu.async_copy(
                    x_slab(c + 1, b), x_bufs[nq * _B + b], isems[nq * _B + b]
                )

        for b in range(_B):
            xb = x_bufs[q * _B + b]
            pltpu.make_async_copy(x_slab(c, b), xb, isems[q * _B + b]).wait()
            pb = pos_bufs[q]

            def rowbody(r, carry, xb=xb, pb=pb):
                for c4 in range(_CPR):
                    sl = pl.ds(c4 * _LANES, _LANES)
                    plsc.addupdate(xb.at[r, sl], pb[r, sl])
                return carry

            lax.fori_loop(0, _R, rowbody, 0)
            pltpu.async_copy(xb, o_slab(c, b), osems[q * _B + b])

    def pair_body(p, carry):
        do_chunk(2 * p, 0)
        do_chunk(2 * p + 1, 1)
        return carry

    lax.fori_loop(0, _NCH // 2, pair_body, 0)

    for b in range(_B):
        pltpu.make_async_copy(x_bufs[b], o_slab(_NCH - 2, b), osems[b]).wait()
        pltpu.make_async_copy(
            x_bufs[_B + b], o_slab(_NCH - 1, b), osems[_B + b]
        ).wait()


def kernel(x, position_embeddings):
    B, S, D = x.shape
    xf = x.reshape(B * S, D)
    mesh = plsc.VectorSubcoreMesh(core_axis_name="c", subcore_axis_name="s")
    f = pl.kernel(
        _sc_body,
        mesh=mesh,
        out_type=jax.ShapeDtypeStruct((B * S, D), jnp.float32),
        scratch_types=(
            [pltpu.VMEM((_R, _D), jnp.float32) for _ in range(10)]
            + [pltpu.SemaphoreType.DMA for _ in range(18)]
        ),
    )
    out = f(xf, position_embeddings)
    return out.reshape(B, S, D)


# speedup vs baseline: 2.5555x; 1.0123x over previous
import functools

import jax
import jax.numpy as jnp
from jax import lax
from jax.experimental import pallas as pl
from jax.experimental.pallas import tpu as pltpu
from jax.experimental.pallas import tpu_sc as plsc

_B, _S, _D = 4, 8192, 768
_NW = 32
_ROWS = _S // _NW
_R = 16
_NCH = _ROWS // _R
_LANES = 16
_CPR = _D // _LANES


def _sc_body(x_hbm, pos_hbm, out_hbm, *refs):
    pos_bufs = refs[0:2]
    x_bufs = refs[2:10]
    psems = refs[10:12]
    isems = refs[12:20]
    osems = refs[20:28]
    wid = lax.axis_index("c") * 16 + lax.axis_index("s")
    row0 = wid * _ROWS

    def pos_slab(c):
        return pos_hbm.at[pl.ds(row0 + c * _R, _R)]

    def x_slab(c, b):
        return x_hbm.at[pl.ds(b * _S + row0 + c * _R, _R)]

    def o_slab(c, b):
        return out_hbm.at[pl.ds(b * _S + row0 + c * _R, _R)]

    pltpu.async_copy(pos_slab(0), pos_bufs[0], psems[0])
    for b in range(_B):
        pltpu.async_copy(x_slab(0, b), x_bufs[b], isems[b])

    def do_chunk(c, q):
        nq = 1 - q
        pltpu.make_async_copy(pos_slab(c), pos_bufs[q], psems[q]).wait()

        @pl.when(c + 1 < _NCH)
        def _():
            pltpu.async_copy(pos_slab(c + 1), pos_bufs[nq], psems[nq])

        for b in range(_B):
            xb = x_bufs[q * _B + b]
            pltpu.make_async_copy(x_slab(c, b), xb, isems[q * _B + b]).wait()

            @pl.when((c > 0) & (c + 1 < _NCH))
            def _():
                pltpu.make_async_copy(
                    x_bufs[nq * _B + b], o_slab(c, b), osems[nq * _B + b]
                ).wait()

            @pl.when(c + 1 < _NCH)
            def _():
                pltpu.async_copy(
                    x_slab(c + 1, b), x_bufs[nq * _B + b], isems[nq * _B + b]
                )

            pb = pos_bufs[q]

            def rowbody(r, carry, xb=xb, pb=pb):
                for c4 in range(_CPR):
                    sl = pl.ds(c4 * _LANES, _LANES)
                    plsc.addupdate(xb.at[r, sl], pb[r, sl])
                return carry

            lax.fori_loop(0, _R, rowbody, 0)
            pltpu.async_copy(xb, o_slab(c, b), osems[q * _B + b])

    def pair_body(p, carry):
        do_chunk(2 * p, 0)
        do_chunk(2 * p + 1, 1)
        return carry

    lax.fori_loop(0, _NCH // 2, pair_body, 0)

    for b in range(_B):
        pltpu.make_async_copy(x_bufs[b], o_slab(_NCH - 2, b), osems[b]).wait()
        pltpu.make_async_copy(
            x_bufs[_B + b], o_slab(_NCH - 1, b), osems[_B + b]
        ).wait()


def kernel(x, position_embeddings):
    B, S, D = x.shape
    xf = x.reshape(B * S, D)
    mesh = plsc.VectorSubcoreMesh(core_axis_name="c", subcore_axis_name="s")
    f = pl.kernel(
        _sc_body,
        mesh=mesh,
        out_type=jax.ShapeDtypeStruct((B * S, D), jnp.float32),
        scratch_types=(
            [pltpu.VMEM((_R, _D), jnp.float32) for _ in range(10)]
            + [pltpu.SemaphoreType.DMA for _ in range(18)]
        ),
    )
    out = f(xf, position_embeddings)
    return out.reshape(B, S, D)
